# baseline (speedup 1.0000x reference)
import jax
import jax.numpy as jnp
from jax.experimental import pallas as pl

_BATCH = 128
_NUM_QUERIES = 1024
_QUERIES_DIM = 512
_B_BLK = 8


def _broadcast_body(w_ref, o_ref):
    o_ref[...] = jnp.broadcast_to(w_ref[...][None], o_ref.shape)


def kernel(queries_weight, batch_size, num_queries):
    del batch_size, num_queries
    return pl.pallas_call(
        _broadcast_body,
        grid=(_BATCH // _B_BLK,),
        in_specs=[
            pl.BlockSpec((_NUM_QUERIES, _QUERIES_DIM), lambda i: (0, 0)),
        ],
        out_specs=pl.BlockSpec(
            (_B_BLK, _NUM_QUERIES, _QUERIES_DIM), lambda i: (i, 0, 0)
        ),
        out_shape=jax.ShapeDtypeStruct(
            (_BATCH, _NUM_QUERIES, _QUERIES_DIM), queries_weight.dtype
        ),
    )(queries_weight)


# speedup vs baseline: 1.0613x; 1.0613x over previous
import jax
import jax.numpy as jnp
from jax.experimental import pallas as pl

_BATCH = 128
_NUM_QUERIES = 1024
_QUERIES_DIM = 512
_B_BLK = 2


def _broadcast_body(w_ref, o_ref):
    o_ref[...] = jnp.broadcast_to(w_ref[...][None], o_ref.shape)


def kernel(queries_weight, batch_size, num_queries):
    del batch_size, num_queries
    return pl.pallas_call(
        _broadcast_body,
        grid=(_BATCH // _B_BLK,),
        in_specs=[
            pl.BlockSpec((_NUM_QUERIES, _QUERIES_DIM), lambda i: (0, 0)),
        ],
        out_specs=pl.BlockSpec(
            (_B_BLK, _NUM_QUERIES, _QUERIES_DIM), lambda i: (i, 0, 0)
        ),
        out_shape=jax.ShapeDtypeStruct(
            (_BATCH, _NUM_QUERIES, _QUERIES_DIM), queries_weight.dtype
        ),
    )(queries_weight)
